# baseline (speedup 1.0000x reference)
import jax
import jax.numpy as jnp
from jax.experimental import pallas as pl
from jax.experimental.pallas import tpu as pltpu

_MAX_REL = 128
_EMB = 64
_LEN = 2048
_TAB = 2 * _MAX_REL + 1
_EXT_PAD = 2 * _LEN
_ROWS_PER_STEP = 8
_STEPS = _LEN // _ROWS_PER_STEP


def _rpe_kernel(w_ref, out_ref, frev_ref, sems):
    k = pl.program_id(0)

    @pl.when(k == 0)
    def _build():
        top = _LEN - _MAX_REL - 1
        frev_ref[0:top, :] = jnp.broadcast_to(
            w_ref[_TAB - 1:_TAB, :], (top, _EMB))
        frev_ref[top + _TAB:_EXT_PAD, :] = jnp.broadcast_to(
            w_ref[0:1, :], (_EXT_PAD - top - _TAB, _EMB))
        for j in range(_TAB):
            frev_ref[top + j:top + j + 1, :] = w_ref[_TAB - 1 - j:_TAB - j, :]

    def copy_for(row, bank, r):
        return pltpu.make_async_copy(
            frev_ref.at[pl.ds(_LEN - 1 - row, _LEN), :],
            out_ref.at[row],
            sems.at[bank, r],
        )

    bank = jax.lax.rem(k, 2)
    for r in range(_ROWS_PER_STEP):
        copy_for(k * _ROWS_PER_STEP + r, bank, r).start()

    @pl.when(k > 0)
    def _wait_prev():
        for r in range(_ROWS_PER_STEP):
            copy_for((k - 1) * _ROWS_PER_STEP + r, 1 - bank, r).wait()

    @pl.when(k == _STEPS - 1)
    def _wait_last():
        for r in range(_ROWS_PER_STEP):
            copy_for(k * _ROWS_PER_STEP + r, bank, r).wait()


@jax.jit
def _run(W):
    return pl.pallas_call(
        _rpe_kernel,
        grid=(_STEPS,),
        in_specs=[pl.BlockSpec((_TAB, _EMB), lambda i: (0, 0))],
        out_specs=pl.BlockSpec(memory_space=pl.ANY),
        out_shape=jax.ShapeDtypeStruct((_LEN, _LEN, _EMB), jnp.float32),
        scratch_shapes=[
            pltpu.VMEM((_EXT_PAD, _EMB), jnp.float32),
            pltpu.SemaphoreType.DMA((2, _ROWS_PER_STEP)),
        ],
    )(W)


def kernel(W, length):
    return _run(W)
